# Initial kernel scaffold; baseline (speedup 1.0000x reference)
import jax
import jax.numpy as jnp
from jax.experimental import pallas as pl


def kernel(input, labels, embed):
    raise NotImplementedError("write your pallas kernel here")


# speedup vs baseline: 2.1825x; 2.1825x over previous
import functools

import jax
import jax.numpy as jnp
from jax import lax
from jax.experimental import pallas as pl
from jax.experimental.pallas import tpu as pltpu
from jax.experimental.pallas import tpu_sc as plsc

_DIM = 32
_N_EMBED = 8192
_N_TOKENS = 64 * 1024
_NC = 2
_NS = 16
_NW = _NC * _NS
_B_PER_W = _N_TOKENS // _NW
_CHUNK = 512
_G = 128

_mesh = plsc.VectorSubcoreMesh(core_axis_name="c", subcore_axis_name="s")


@functools.partial(
    pl.kernel,
    out_type=(
        jax.ShapeDtypeStruct((_N_TOKENS, _DIM), jnp.float32),
        jax.ShapeDtypeStruct((_NW, 16), jnp.float32),
    ),
    mesh=_mesh,
    scratch_types=[
        pltpu.VMEM((_B_PER_W,), jnp.int32),
        pltpu.VMEM((_CHUNK, _DIM), jnp.float32),
        pltpu.VMEM((_CHUNK, _DIM), jnp.float32),
        pltpu.VMEM((1, 16), jnp.float32),
        pltpu.SemaphoreType.DMA,
    ],
    compiler_params=pltpu.CompilerParams(use_tc_tiling_on_sc=False),
)
def _vq_lookup(inp_hbm, lab_hbm, emb_hbm, quant_hbm, part_hbm,
               idx_v, rows_v, inp_v, acc_v, sem):
    wid = lax.axis_index("s") * _NC + lax.axis_index("c")
    base = wid * _B_PER_W
    acc_v[...] = jnp.zeros((1, 16), jnp.float32)
    pltpu.sync_copy(lab_hbm.at[pl.ds(base, _B_PER_W)], idx_v)
    for c in range(_B_PER_W // _CHUNK):
        off = c * _CHUNK
        copies = [
            pltpu.async_copy(
                emb_hbm.at[idx_v.at[pl.ds(off + j * _G, _G)]],
                rows_v.at[pl.ds(j * _G, _G)],
                sem,
            )
            for j in range(_CHUNK // _G)
        ]
        pltpu.sync_copy(inp_hbm.at[pl.ds(base + off, _CHUNK)], inp_v)
        for cp in copies:
            cp.wait()
        pltpu.sync_copy(rows_v, quant_hbm.at[pl.ds(base + off, _CHUNK)])

        @pl.loop(0, _CHUNK)
        def _(i):
            d0 = (rows_v.at[pl.ds(i, 1), pl.ds(0, 16)][...]
                  - inp_v.at[pl.ds(i, 1), pl.ds(0, 16)][...])
            d1 = (rows_v.at[pl.ds(i, 1), pl.ds(16, 16)][...]
                  - inp_v.at[pl.ds(i, 1), pl.ds(16, 16)][...])
            acc_v[...] = acc_v[...] + d0 * d0 + d1 * d1

    pltpu.sync_copy(acc_v, part_hbm.at[pl.ds(wid, 1)])


def kernel(input, labels, embed):
    inp_flat = input.reshape(_N_TOKENS, _DIM)
    emb_t = embed.T
    quant, partials = _vq_lookup(inp_flat, labels, emb_t)
    quantize = quant.reshape(input.shape)
    diff = jnp.sum(partials) / jnp.float32(_N_TOKENS * _DIM)
    embed_ind = labels.reshape(input.shape[:-1])
    return quantize, diff, embed_ind
